# baseline (speedup 1.0000x reference)
import jax
import jax.numpy as jnp
from jax import lax
from jax.experimental import pallas as pl
from jax.experimental.pallas import tpu as pltpu

B = 8
SIZE = 28
P = SIZE * SIZE
L = 256
D = 10000
DP = 10240
CD = 2048
NCHUNK = DP // CD
NC = 10


def _body(xf_ref, px_ref, py_ref, lev_ref, w_ref, out_ref):
    idx = jnp.clip(jnp.round(xf_ref[...] * (L - 1)), 0, L - 1).astype(
        jnp.int32)
    lgrid = lax.broadcasted_iota(jnp.int32, (B, L, P), 1)
    oh = (idx[:, None, :] == lgrid).astype(jnp.bfloat16).reshape(B * L, P)

    def step(c, part):
        sl = pl.ds(c * CD, CD)
        s = py_ref[:, sl][:, None, :] + px_ref[:, sl][None, :, :]
        posmat = jnp.where(s > 0, 1.0, -1.0).astype(jnp.bfloat16).reshape(
            P, CD)
        acc = lax.dot_general(
            oh, posmat,
            dimension_numbers=(((1,), (0,)), ((), ())),
            preferred_element_type=jnp.float32,
        ).reshape(B, L, CD)
        ms = jnp.sum(acc * lev_ref[:, sl][None, :, :], axis=1)
        enc = jnp.where(ms > 0, 1.0, -1.0).astype(jnp.float32)
        return part + lax.dot_general(
            enc, w_ref[:, sl],
            dimension_numbers=(((1,), (1,)), ((), ())),
            preferred_element_type=jnp.float32,
        )

    out_ref[...] = lax.fori_loop(0, NCHUNK, step,
                                 jnp.zeros((B, NC), jnp.float32))


def kernel(x, pos_x, pos_y, levels, W):
    xf = x.reshape(B, P)
    pad = DP - D
    px = jnp.pad(pos_x, ((0, 0), (0, pad)))
    py = jnp.pad(pos_y, ((0, 0), (0, pad)))
    lev = jnp.pad(levels, ((0, 0), (0, pad)))
    wp = jnp.pad(W, ((0, 0), (0, pad)))
    return pl.pallas_call(
        _body,
        out_shape=jax.ShapeDtypeStruct((B, NC), jnp.float32),
    )(xf, px, py, lev, wp)


# speedup vs baseline: 1.4569x; 1.4569x over previous
import jax
import jax.numpy as jnp
from jax import lax
from jax.experimental import pallas as pl
from jax.experimental.pallas import tpu as pltpu

B = 8
SIZE = 28
P = SIZE * SIZE
L = 256
D = 10000
CD = 2560
NCHUNK = (D + CD - 1) // CD
DP = NCHUNK * CD
NC = 10


def _body(xf_ref, px_ref, py_ref, lev_ref, w_ref, out_ref, oh_ref, enc_ref):
    g = pl.program_id(0)

    dmask = (g * CD + lax.broadcasted_iota(jnp.int32, (1, CD), 1)) < D

    @pl.when(g == 0)
    def _():
        idx = jnp.clip(jnp.round(xf_ref[...] * (L - 1)), 0, L - 1).astype(
            jnp.int32)
        lgrid = lax.broadcasted_iota(jnp.int32, (B, L, P), 1)
        oh_ref[...] = (idx[:, None, :] == lgrid).astype(
            jnp.bfloat16).reshape(B * L, P)

    s = py_ref[...][:, None, :] + px_ref[...][None, :, :]
    posmat = jnp.where(s > 0, 1.0, -1.0).astype(jnp.bfloat16).reshape(P, CD)

    acc = lax.dot_general(
        oh_ref[...], posmat,
        dimension_numbers=(((1,), (0,)), ((), ())),
        preferred_element_type=jnp.float32,
    ).reshape(B, L, CD)

    lev = jnp.where(dmask, lev_ref[...], 0.0)
    ms = jnp.sum(acc * lev[None, :, :], axis=1)
    enc_ref[:, pl.ds(g * CD, CD)] = jnp.where(ms > 0, 1.0, -1.0).astype(
        jnp.float32)

    @pl.when(g == NCHUNK - 1)
    def _():
        out_ref[...] = lax.dot_general(
            enc_ref[...], w_ref[...],
            dimension_numbers=(((1,), (1,)), ((), ())),
            preferred_element_type=jnp.float32,
        )


def kernel(x, pos_x, pos_y, levels, W):
    xf = x.reshape(B, P)
    wp = jnp.pad(W, ((0, 0), (0, DP - D)))
    return pl.pallas_call(
        _body,
        grid=(NCHUNK,),
        in_specs=[
            pl.BlockSpec((B, P), lambda g: (0, 0)),
            pl.BlockSpec((SIZE, CD), lambda g: (0, g)),
            pl.BlockSpec((SIZE, CD), lambda g: (0, g)),
            pl.BlockSpec((L, CD), lambda g: (0, g)),
            pl.BlockSpec((NC, DP), lambda g: (0, 0)),
        ],
        out_specs=pl.BlockSpec((B, NC), lambda g: (0, 0)),
        out_shape=jax.ShapeDtypeStruct((B, NC), jnp.float32),
        scratch_shapes=[
            pltpu.VMEM((B * L, P), jnp.bfloat16),
            pltpu.VMEM((B, DP), jnp.float32),
        ],
        compiler_params=pltpu.CompilerParams(
            dimension_semantics=("arbitrary",),
        ),
    )(xf, pos_x, pos_y, levels, wp)


# speedup vs baseline: 1.4805x; 1.0162x over previous
import jax
import jax.numpy as jnp
from jax import lax
from jax.experimental import pallas as pl
from jax.experimental.pallas import tpu as pltpu

B = 8
SIZE = 28
P = SIZE * SIZE
L = 256
D = 10000
CD = 2048
NCHUNK = (D + CD - 1) // CD
DP = NCHUNK * CD
NC = 10


def _body(xf_ref, px_ref, py_ref, lev_ref, w_ref, out_ref, oh_ref, enc_ref):
    g = pl.program_id(0)

    dmask = (g * CD + lax.broadcasted_iota(jnp.int32, (1, CD), 1)) < D

    @pl.when(g == 0)
    def _():
        idx = jnp.clip(jnp.round(xf_ref[...] * (L - 1)), 0, L - 1).astype(
            jnp.int32)
        lgrid = lax.broadcasted_iota(jnp.int32, (B, L, P), 1)
        oh_ref[...] = (idx[:, None, :] == lgrid).astype(
            jnp.bfloat16).reshape(B * L, P)

    s = py_ref[...][:, None, :] + px_ref[...][None, :, :]
    posmat = jnp.where(s > 0, 1.0, -1.0).astype(jnp.bfloat16).reshape(P, CD)

    acc = lax.dot_general(
        oh_ref[...], posmat,
        dimension_numbers=(((1,), (0,)), ((), ())),
        preferred_element_type=jnp.float32,
    ).reshape(B, L, CD)

    lev = jnp.where(dmask, lev_ref[...], 0.0)
    ms = jnp.sum(acc * lev[None, :, :], axis=1)
    enc_ref[:, pl.ds(g * CD, CD)] = jnp.where(ms > 0, 1.0, -1.0).astype(
        jnp.float32)

    @pl.when(g == NCHUNK - 1)
    def _():
        out_ref[...] = lax.dot_general(
            enc_ref[...], w_ref[...],
            dimension_numbers=(((1,), (1,)), ((), ())),
            preferred_element_type=jnp.float32,
        )


def kernel(x, pos_x, pos_y, levels, W):
    xf = x.reshape(B, P)
    wp = jnp.pad(W, ((0, 0), (0, DP - D)))
    return pl.pallas_call(
        _body,
        grid=(NCHUNK,),
        in_specs=[
            pl.BlockSpec((B, P), lambda g: (0, 0)),
            pl.BlockSpec((SIZE, CD), lambda g: (0, g)),
            pl.BlockSpec((SIZE, CD), lambda g: (0, g)),
            pl.BlockSpec((L, CD), lambda g: (0, g)),
            pl.BlockSpec((NC, DP), lambda g: (0, 0)),
        ],
        out_specs=pl.BlockSpec((B, NC), lambda g: (0, 0)),
        out_shape=jax.ShapeDtypeStruct((B, NC), jnp.float32),
        scratch_shapes=[
            pltpu.VMEM((B * L, P), jnp.bfloat16),
            pltpu.VMEM((B, DP), jnp.float32),
        ],
        compiler_params=pltpu.CompilerParams(
            dimension_semantics=("arbitrary",),
        ),
    )(xf, pos_x, pos_y, levels, wp)


# speedup vs baseline: 1.5021x; 1.0146x over previous
import jax
import jax.numpy as jnp
from jax import lax
from jax.experimental import pallas as pl
from jax.experimental.pallas import tpu as pltpu

B = 8
SIZE = 28
P = SIZE * SIZE
L = 256
D = 10000
CD = 2048
NCHUNK = (D + CD - 1) // CD
NC = 10


def _body(xf_ref, px_ref, py_ref, lev_ref, w_ref, out_ref, oh_ref):
    g = pl.program_id(0)

    dmask = (g * CD + lax.broadcasted_iota(jnp.int32, (1, CD), 1)) < D

    @pl.when(g == 0)
    def _():
        idx = jnp.clip(jnp.round(xf_ref[...] * (L - 1)), 0, L - 1).astype(
            jnp.int32)
        lgrid = lax.broadcasted_iota(jnp.int32, (B, L, P), 1)
        oh_ref[...] = (idx[:, None, :] == lgrid).astype(
            jnp.bfloat16).reshape(B * L, P)

    oh = oh_ref[...]

    pxb = px_ref[...].astype(jnp.bfloat16)
    pyb = py_ref[...].astype(jnp.bfloat16)
    s = pyb[:, None, :] + pxb[None, :, :]
    one = jnp.bfloat16(1.0)
    posmat = jnp.where(s > 0, one, -one).reshape(P, CD)

    acc = lax.dot_general(
        oh, posmat,
        dimension_numbers=(((1,), (0,)), ((), ())),
        preferred_element_type=jnp.float32,
    ).reshape(B, L, CD)

    lev = jnp.where(dmask, lev_ref[...], 0.0)
    ms = jnp.sum(acc * lev[None, :, :], axis=1)
    enc = jnp.where(ms > 0, 1.0, -1.0).astype(jnp.float32)

    wm = jnp.where(dmask, w_ref[...], 0.0)
    part = lax.dot_general(
        enc, wm,
        dimension_numbers=(((1,), (1,)), ((), ())),
        preferred_element_type=jnp.float32,
    )

    @pl.when(g == 0)
    def _():
        out_ref[...] = jnp.zeros_like(out_ref)

    out_ref[...] += part


def kernel(x, pos_x, pos_y, levels, W):
    xf = x.reshape(B, P)
    return pl.pallas_call(
        _body,
        grid=(NCHUNK,),
        in_specs=[
            pl.BlockSpec((B, P), lambda g: (0, 0)),
            pl.BlockSpec((SIZE, CD), lambda g: (0, g)),
            pl.BlockSpec((SIZE, CD), lambda g: (0, g)),
            pl.BlockSpec((L, CD), lambda g: (0, g)),
            pl.BlockSpec((NC, CD), lambda g: (0, g)),
        ],
        out_specs=pl.BlockSpec((B, NC), lambda g: (0, 0)),
        out_shape=jax.ShapeDtypeStruct((B, NC), jnp.float32),
        scratch_shapes=[pltpu.VMEM((B * L, P), jnp.bfloat16)],
        compiler_params=pltpu.CompilerParams(
            dimension_semantics=("arbitrary",),
        ),
    )(xf, pos_x, pos_y, levels, W)
